# baseline (speedup 1.0000x reference)
import functools

import jax
import jax.numpy as jnp
from jax.experimental import pallas as pl
from jax.experimental.pallas import tpu as pltpu

_BF = jnp.bfloat16
_F32 = jnp.float32


def _trunc16(a):
    ai = jax.lax.bitcast_convert_type(a, jnp.int32)
    return jax.lax.bitcast_convert_type(ai & jnp.int32(-65536), _F32)


def _split3(a):
    af = _trunc16(a)
    r1 = a - af
    bf = _trunc16(r1)
    r2 = r1 - bf
    return af.astype(_BF), bf.astype(_BF), r2.astype(_BF)


def _dot(a, b):
    return jax.lax.dot_general(a, b, (((1,), (0,)), ((), ())),
                               preferred_element_type=_F32)


def _bdot(a, b_ref):
    return _dot(a.astype(_BF), b_ref[...])


def _body(x_ref, w1_ref, w2_ref, e2_ref, e2ta_ref, e2tb_ref, e2tc_ref,
          w3_ref, w4_ref,
          b1_ref, b2_ref, b3_ref, b4_ref, e2c_ref,
          recon_ref, ze_ref, embout_ref, *, K, P, F):
    h1 = jnp.maximum(_bdot(x_ref[...], w1_ref) + b1_ref[...], 0.0)
    ze = _bdot(h1, w2_ref) + b2_ref[...]
    ze_ref[...] = ze

    scores = e2c_ref[...] - 2.0 * _bdot(ze, e2_ref)

    iota = jax.lax.broadcasted_iota(jnp.int32, (scores.shape[0], K), 1)
    ohs = []
    for p in range(P):
        s = scores[:, p * K:(p + 1) * K]
        m = jnp.min(s, axis=1, keepdims=True)
        cand = jnp.where(s == m, iota, K)
        kmin = jnp.min(cand, axis=1, keepdims=True)
        ohs.append((iota == kmin).astype(_BF))
    oh = jnp.concatenate(ohs, axis=1)
    zq = ((_dot(oh, e2ta_ref[...]) + _dot(oh, e2tb_ref[...]))
          + _dot(oh, e2tc_ref[...]))
    embout_ref[...] = zq

    h3 = jnp.maximum(_bdot(zq, w3_ref) + b3_ref[...], 0.0)
    recon_ref[...] = jax.nn.sigmoid(_bdot(h3, w4_ref) + b4_ref[...])


def kernel(x, W1, b1, W2, b2, W3, b3, W4, b4, emb_weight):
    B, L = x.shape
    D, K = emb_weight.shape
    H = W2.shape[0]
    P = H // D
    F1 = W1.shape[0]
    F = 512
    BB = 256

    def padto(a, rows, cols):
        return jnp.pad(a, ((0, rows - a.shape[0]), (0, cols - a.shape[1])))

    W1b = padto(W1.T, L, F).astype(_BF)
    W2b = padto(W2.T, F, H).astype(_BF)
    W3b = padto(W3.T, H, F).astype(_BF)
    W4b = padto(W4.T, F, L).astype(_BF)

    eye = jnp.eye(P, dtype=_F32)
    E2 = (emb_weight[:, None, None, :] * eye[None, :, :, None]
          ).reshape(D * P, P * K)
    E2T = (eye[:, None, None, :] * emb_weight.T[None, :, :, None]
           ).reshape(P * K, D * P)
    E2b = E2.astype(_BF)
    E2Ta, E2Tb, E2Tc = _split3(E2T)
    e2c = jnp.sum(emb_weight * emb_weight, axis=0)
    e2c = jnp.tile(e2c, P)

    b1r = jnp.pad(b1, (0, F - F1)).reshape(1, F)
    b2r = b2.reshape(1, H)
    b3r = jnp.pad(b3, (0, F - F1)).reshape(1, F)
    b4r = b4.reshape(1, L)
    e2cr = e2c.reshape(1, P * K)

    grid = (B // BB,)
    full = lambda shape: pl.BlockSpec(shape, lambda i: (0, 0))
    row = lambda shape: pl.BlockSpec(shape, lambda i: (i, 0))

    recon, ze, embout = pl.pallas_call(
        functools.partial(_body, K=K, P=P, F=F),
        grid=grid,
        in_specs=[
            row((BB, L)),
            full((L, F)),
            full((F, H)),
            full((H, P * K)),
            full((P * K, H)), full((P * K, H)), full((P * K, H)),
            full((H, F)), full((F, L)),
            full((1, F)), full((1, H)), full((1, F)), full((1, L)),
            full((1, P * K)),
        ],
        out_specs=(row((BB, L)), row((BB, H)), row((BB, H))),
        out_shape=(
            jax.ShapeDtypeStruct((B, L), x.dtype),
            jax.ShapeDtypeStruct((B, H), x.dtype),
            jax.ShapeDtypeStruct((B, H), x.dtype),
        ),
        compiler_params=pltpu.CompilerParams(
            dimension_semantics=("arbitrary",)),
    )(x, W1b, W2b, E2b, E2Ta, E2Tb, E2Tc, W3b, W4b,
      b1r, b2r, b3r, b4r, e2cr)

    return recon, ze.reshape(B, D, P), embout
